# baseline (speedup 1.0000x reference)
import functools

import numpy as np

import jax
import jax.numpy as jnp
from jax import lax
from jax.experimental import pallas as pl
from jax.experimental.pallas import tpu as pltpu
from jax.experimental.pallas import tpu_sc as plsc

N = 10000
E = 320000
H = 128
OUT_NODES = 3
OUT_GLOB = 4

NC = 2
NS = 16
NW = NC * NS
EPW = E // NW
CH_G = 80
CH_S = 40
NB = 5
EC_LIST = (192000, 128000)
K = len(EC_LIST)
NCHUNK_CNT = EPW // CH_G

TN = 2000
TE = 2000

_f32 = jnp.float32
_bf16 = jnp.bfloat16
HP = H // 2


def _leaky(v):
    return jnp.where(v >= 0, v, 0.01 * v)


def _dot(a, b):
    return jnp.dot(a, b, preferred_element_type=_f32)


def _node_enc_body(x_ref, w1, b1, w2, b2, wa, wb, nf_ref, p_ref, q_ref):
    h = _leaky(_dot(x_ref[...], w1[...]) + b1[...])
    nf = _dot(h, w2[...]) + b2[...]
    nf_ref[...] = nf
    p_ref[...] = _dot(nf, wa[...])
    q_ref[...] = _dot(nf, wb[...])


def _bdot(a, b):
    return jnp.dot(a.astype(_bf16), b.astype(_bf16),
                   preferred_element_type=_f32)


def _edge_blk0_body(pq_ref, ea_ref, we1, be1, we2, be2,
                    wc, b1, w2, b2, eout_ref):
    he = _leaky(_bdot(ea_ref[...], we1[...]) + be1[...])
    ef = _bdot(he, we2[...]) + be2[...]
    pre = pq_ref[...] + _bdot(ef, wc[...]) + b1[...]
    eout_ref[...] = _bdot(_leaky(pre), w2[...]) + b2[...]


def _edge_blk1_body(pq_ref, ea_ref, eo0_ref, we1, be1, we2, be2,
                    wc, b1, w2, b2, eout_ref):
    he = _leaky(_bdot(ea_ref[...], we1[...]) + be1[...])
    ef = _bdot(he, we2[...]) + be2[...] + eo0_ref[...]
    pre = pq_ref[...] + _bdot(ef, wc[...]) + b1[...]
    eout_ref[...] = _bdot(_leaky(pre), w2[...]) + b2[...]


def _node_blk_body(nf_ref, *args):
    s_refs = args[:K]
    (c_ref, wa, wb, b1, w2, b2, wpa, wpb,
     nfn_ref, p_ref, q_ref) = args[K:]
    nf = nf_ref[...]
    s = s_refs[0][0] + s_refs[0][1]
    for sr in s_refs[1:]:
        s = s + sr[0] + sr[1]
    cnt = c_ref[0][:, 0:1] + c_ref[1][:, 0:1]
    aggr = s / jnp.maximum(cnt, 1.0)
    h = _leaky(_dot(nf, wa[...]) + _dot(aggr, wb[...]) + b1[...])
    nfn = _dot(h, w2[...]) + b2[...] + nf
    nfn_ref[...] = nfn
    if p_ref is not None:
        p_ref[...] = _dot(nfn, wpa[...])
        q_ref[...] = _dot(nfn, wpb[...])


def _node_blk_last_body(nf_ref, *args):
    s_refs = args[:K]
    c_ref, wa, wb, b1, w2, b2, nfn_ref = args[K:]
    _node_blk_body(nf_ref, *s_refs, c_ref, wa, wb, b1, w2, b2, None, None,
                   nfn_ref, None, None)


def _decode_body(nf_ref, t_ref, wd1, bd1, wd2, bd2,
                 wg1, bg1, wg2, bg2, wg3, bg3,
                 y_ref, g_ref, m_sc, s_sc, v_sc):
    i = pl.program_id(0)
    nf = nf_ref[...]
    h = _leaky(_dot(nf, wd1[...]) + bd1[...])
    y_ref[...] = _dot(h, wd2[...]) + bd2[...]

    alpha = nf * t_ref[...]
    tmax = jnp.max(alpha, axis=0, keepdims=True)

    @pl.when(i == 0)
    def _():
        e = jnp.exp(alpha - tmax)
        m_sc[...] = tmax
        s_sc[...] = jnp.sum(e, axis=0, keepdims=True)
        v_sc[...] = jnp.sum(e * nf, axis=0, keepdims=True)

    @pl.when(i > 0)
    def _():
        m_old = m_sc[...]
        m_new = jnp.maximum(m_old, tmax)
        scale = jnp.exp(m_old - m_new)
        e = jnp.exp(alpha - m_new)
        m_sc[...] = m_new
        s_sc[...] = s_sc[...] * scale + jnp.sum(e, axis=0, keepdims=True)
        v_sc[...] = v_sc[...] * scale + jnp.sum(e * nf, axis=0, keepdims=True)

    @pl.when(i == pl.num_programs(0) - 1)
    def _():
        gin = v_sc[...] / s_sc[...]
        g = _leaky(_dot(gin, wg1[...]) + bg1[...])
        g = _leaky(_dot(g, wg2[...]) + bg2[...])
        g_ref[...] = _dot(g, wg3[...]) + bg3[...]


def _full(shape):
    return pl.BlockSpec(shape, lambda i: (0,) * len(shape))


def _rows(tile, width):
    return pl.BlockSpec((tile, width), lambda i: (i, 0))


def _node_encode(x, w1, b1, w2, b2, wa, wb):
    return pl.pallas_call(
        _node_enc_body,
        grid=(N // TN,),
        in_specs=[_rows(TN, H), _full((H, H)), _full((1, H)), _full((H, H)),
                  _full((1, H)), _full((H, H)), _full((H, H))],
        out_specs=[_rows(TN, H)] * 3,
        out_shape=[jax.ShapeDtypeStruct((N, H), _f32)] * 3,
    )(x, w1, b1, w2, b2, wa, wb)


def _edge_block0(pq, ea, we1, be1, we2, be2, wc, b1, w2, b2):
    f_in = ea.shape[1]
    rows = pq.shape[0]
    return pl.pallas_call(
        _edge_blk0_body,
        grid=(rows // TE,),
        in_specs=[_rows(TE, H), _rows(TE, f_in),
                  _full((f_in, H)), _full((1, H)), _full((H, H)),
                  _full((1, H)), _full((H, H)), _full((1, H)),
                  _full((H, H)), _full((1, H))],
        out_specs=_rows(TE, H),
        out_shape=jax.ShapeDtypeStruct((rows, H), _f32),
    )(pq, ea, we1, be1, we2, be2, wc, b1, w2, b2)


def _edge_block1(pq, ea, eo0, we1, be1, we2, be2, wc, b1, w2, b2):
    f_in = ea.shape[1]
    rows = pq.shape[0]
    return pl.pallas_call(
        _edge_blk1_body,
        grid=(rows // TE,),
        in_specs=[_rows(TE, H), _rows(TE, f_in), _rows(TE, H),
                  _full((f_in, H)), _full((1, H)), _full((H, H)),
                  _full((1, H)), _full((H, H)), _full((1, H)),
                  _full((H, H)), _full((1, H))],
        out_specs=_rows(TE, H),
        out_shape=jax.ShapeDtypeStruct((rows, H), _f32),
    )(pq, ea, eo0, we1, be1, we2, be2, wc, b1, w2, b2)


def _node_block(nf, sums_list, cnt, wa, wb, b1, w2, b2, wpa=None, wpb=None):
    sums_spec = pl.BlockSpec((NC, TN, H), lambda i: (0, i, 0))
    common = ([_rows(TN, H)] + [sums_spec] * K + [sums_spec, _full((H, H)),
              _full((H, H)), _full((1, H)), _full((H, H)), _full((1, H))])
    if wpa is None:
        return pl.pallas_call(
            _node_blk_last_body,
            grid=(N // TN,),
            in_specs=common,
            out_specs=_rows(TN, H),
            out_shape=jax.ShapeDtypeStruct((N, H), _f32),
        )(nf, *sums_list, cnt, wa, wb, b1, w2, b2)
    return pl.pallas_call(
        _node_blk_body,
        grid=(N // TN,),
        in_specs=common + [_full((H, H)), _full((H, H))],
        out_specs=[_rows(TN, H)] * 3,
        out_shape=[jax.ShapeDtypeStruct((N, H), _f32)] * 3,
    )(nf, *sums_list, cnt, wa, wb, b1, w2, b2, wpa, wpb)


def _decode(nf, t_row, wd1, bd1, wd2, bd2, wg1, bg1, wg2, bg2, wg3, bg3):
    return pl.pallas_call(
        _decode_body,
        grid=(N // TN,),
        in_specs=[_rows(TN, H), _full((1, H)),
                  _full((H, H)), _full((1, H)), _full((H, OUT_NODES)),
                  _full((1, OUT_NODES)),
                  _full((H, H)), _full((1, H)), _full((H, H)), _full((1, H)),
                  _full((H, OUT_GLOB)), _full((1, OUT_GLOB))],
        out_specs=[_rows(TN, OUT_NODES),
                   pl.BlockSpec((1, OUT_GLOB), lambda i: (0, 0))],
        out_shape=[jax.ShapeDtypeStruct((N, OUT_NODES), _f32),
                   jax.ShapeDtypeStruct((1, OUT_GLOB), _f32)],
        scratch_shapes=[pltpu.VMEM((1, H), _f32)] * 3,
    )(nf, t_row, wd1, bd1, wd2, bd2, wg1, bg1, wg2, bg2, wg3, bg3)


def _ring(nchunk, load_start, load_wait, store_start, store_wait):
    nb = NB
    for k in range(nb - 1):
        load_start(k, k)

    @pl.loop(0, nchunk - nb, step=nb)
    def _(base):
        for b in range(nb):
            i = base + b
            load_wait(i, b)
            store_start(i, b)
            s_n = (b + nb - 1) % nb

            def _drain():
                store_wait(i, s_n)

            if b == 0:
                pl.when(base > 0)(_drain)
            else:
                _drain()
            load_start(i + nb - 1, s_n)

    i0 = nchunk - nb
    load_wait(i0, 0)
    store_start(i0, 0)
    store_wait(i0, nb - 1)
    load_start(nchunk - 1, nb - 1)
    for b in range(1, nb):
        load_wait(i0 + b, b)
        store_start(i0 + b, b)
    for b in range(nb):
        store_wait(i0 + b, b)


@functools.lru_cache(maxsize=None)
def _mesh():
    return plsc.VectorSubcoreMesh(core_axis_name="c", subcore_axis_name="s",
                                  num_cores=NC, num_subcores=NS)


@functools.lru_cache(maxsize=None)
def _sc_slice_kernels(ec):
    mesh = _mesh()
    epw_c = ec // NW
    nchunk_gc = epw_c // CH_G
    nchunk_sc = epw_c // CH_S

    @functools.partial(
        pl.kernel,
        out_type=jax.ShapeDtypeStruct((ec, H), _f32),
        mesh=mesh,
        scratch_types=[pltpu.VMEM((nchunk_gc, CH_G), jnp.int32)] * 2
        + [pltpu.VMEM((CH_G, H), _f32)] * (2 * NB)
        + [pltpu.SemaphoreType.DMA] * (3 * NB),
    )
    def _sc_gather2(dst_hbm, src_hbm, p_hbm, q_hbm, pq_hbm, *rest):
        idxd = rest[0]
        idxs = rest[1]
        bufp = rest[2:2 + NB]
        bufq = rest[2 + NB:2 + 2 * NB]
        sem_p = rest[2 + 2 * NB:2 + 3 * NB]
        sem_q = rest[2 + 3 * NB:2 + 4 * NB]
        sem_w = rest[2 + 4 * NB:2 + 5 * NB]
        wid = lax.axis_index("s") * NC + lax.axis_index("c")
        wbase = wid * epw_c
        pltpu.sync_copy(dst_hbm.at[wid], idxd)
        pltpu.sync_copy(src_hbm.at[wid], idxs)

        def out_sl(i):
            return pq_hbm.at[pl.ds(wbase + i * CH_G, CH_G)]

        def load_start(i, sl):
            pltpu.async_copy(p_hbm.at[idxd.at[i]], bufp[sl], sem_p[sl])
            pltpu.async_copy(q_hbm.at[idxs.at[i]], bufq[sl], sem_q[sl])

        def load_wait(i, sl):
            pltpu.make_async_copy(p_hbm.at[idxd.at[i]], bufp[sl],
                                  sem_p[sl]).wait()
            pltpu.make_async_copy(q_hbm.at[idxs.at[i]], bufq[sl],
                                  sem_q[sl]).wait()

        def w_start(i, sl):
            @pl.loop(0, CH_G)
            def _(r):
                for c in range(H // 16):
                    plsc.addupdate(bufp[sl].at[r, pl.ds(c * 16, 16)],
                                   bufq[sl][r, pl.ds(c * 16, 16)])

            pltpu.async_copy(bufp[sl], out_sl(i), sem_w[sl])

        def w_wait(i, sl):
            pltpu.make_async_copy(bufp[sl], out_sl(i), sem_w[sl]).wait()

        _ring(nchunk_gc, load_start, load_wait, w_start, w_wait)

    @functools.partial(
        pl.kernel,
        out_type=jax.ShapeDtypeStruct((NC, N, H), _f32),
        mesh=mesh,
        scratch_types=[pltpu.VMEM((nchunk_sc, CH_S), jnp.int32)]
        + [pltpu.VMEM((CH_S, H), _f32)] * NB
        + [pltpu.SemaphoreType.DMA] * (2 * NB)
        + [pltpu.VMEM_SHARED((N, H), _f32)],
    )
    def _sc_scatter(dst_hbm, eout_hbm, zeros_hbm, out_hbm, *rest):
        idx_vm = rest[0]
        bufs = rest[1:1 + NB]
        sem_r = rest[1 + NB:1 + 2 * NB]
        sem_a = rest[1 + 2 * NB:1 + 3 * NB]
        acc_sh = rest[1 + 3 * NB]
        cid = lax.axis_index("c")
        sid = lax.axis_index("s")

        @pl.when(sid == 0)
        def _():
            pltpu.sync_copy(zeros_hbm, acc_sh)

        wid = sid * NC + cid
        wbase = wid * epw_c
        pltpu.sync_copy(dst_hbm.at[wid], idx_vm)
        plsc.subcore_barrier()

        def eo_sl(i):
            return eout_hbm.at[pl.ds(wbase + i * CH_S, CH_S)]

        def load_start(i, sl):
            pltpu.async_copy(eo_sl(i), bufs[sl], sem_r[sl])

        def load_wait(i, sl):
            pltpu.make_async_copy(eo_sl(i), bufs[sl], sem_r[sl]).wait()

        def a_start(i, sl):
            pltpu.async_copy(bufs[sl], acc_sh.at[idx_vm.at[i]], sem_a[sl],
                             add=True)

        def a_wait(i, sl):
            pltpu.make_async_copy(bufs[sl], acc_sh.at[idx_vm.at[i]],
                                  sem_a[sl]).wait()

        _ring(nchunk_sc, load_start, load_wait, a_start, a_wait)

        plsc.subcore_barrier()
        rows = (N // NS) // 8 * 8
        rstart = sid * rows
        pltpu.sync_copy(acc_sh.at[pl.ds(rstart, rows)],
                        out_hbm.at[cid].at[pl.ds(rstart, rows)])

        @pl.when(sid == NS - 1)
        def _():
            pltpu.sync_copy(acc_sh.at[pl.ds(NS * rows, N - NS * rows)],
                            out_hbm.at[cid].at[pl.ds(NS * rows, N - NS * rows)])

    return _sc_gather2, _sc_scatter


@functools.lru_cache(maxsize=None)
def _sc_counts_kernel():
    mesh = _mesh()

    @functools.partial(
        pl.kernel,
        out_type=jax.ShapeDtypeStruct((NC, N, H), _f32),
        mesh=mesh,
        scratch_types=[pltpu.VMEM((NCHUNK_CNT, CH_G), jnp.int32),
                       pltpu.VMEM((CH_G, H), _f32),
                       pltpu.SemaphoreType.DMA,
                       pltpu.VMEM_SHARED((N, H), _f32)],
    )
    def _sc_counts(dst_hbm, ones_hbm, zeros_hbm, out_hbm,
                   idx_vm, ones_v, sem_a, acc_sh):
        cid = lax.axis_index("c")
        sid = lax.axis_index("s")

        @pl.when(sid == 0)
        def _():
            pltpu.sync_copy(zeros_hbm, acc_sh)

        wid = sid * NC + cid
        pltpu.sync_copy(dst_hbm.at[wid], idx_vm)
        pltpu.sync_copy(ones_hbm, ones_v)
        plsc.subcore_barrier()
        DEPTH = 8

        @pl.loop(0, NCHUNK_CNT)
        def _(i):
            @pl.when(i >= DEPTH)
            def _():
                pltpu.make_async_copy(ones_v, acc_sh.at[idx_vm.at[i]],
                                      sem_a).wait()

            pltpu.async_copy(ones_v, acc_sh.at[idx_vm.at[i]], sem_a, add=True)

        @pl.loop(0, DEPTH)
        def _(i):
            pltpu.make_async_copy(ones_v, acc_sh.at[idx_vm.at[i]],
                                  sem_a).wait()

        plsc.subcore_barrier()
        rows = (N // NS) // 8 * 8
        rstart = sid * rows
        pltpu.sync_copy(acc_sh.at[pl.ds(rstart, rows)],
                        out_hbm.at[cid].at[pl.ds(rstart, rows)])

        @pl.when(sid == NS - 1)
        def _():
            pltpu.sync_copy(acc_sh.at[pl.ds(NS * rows, N - NS * rows)],
                            out_hbm.at[cid].at[pl.ds(NS * rows, N - NS * rows)])

    return _sc_counts


def _row(b):
    return b.reshape(1, -1)


def kernel(x, edge_index, edge_attr, batch, params):
    del batch
    src_f = edge_index[0]
    dst_f = edge_index[1]
    dst_cnt = dst_f.reshape(NW, NCHUNK_CNT, CH_G)
    offs = [0]
    for ec in EC_LIST:
        offs.append(offs[-1] + ec)
    src_c, dst_c, dst_cs, ea_c = [], [], [], []
    for k, ec in enumerate(EC_LIST):
        o = offs[k]
        src_c.append(src_f[o:o + ec].reshape(NW, ec // NW // CH_G, CH_G))
        dst_c.append(dst_f[o:o + ec].reshape(NW, ec // NW // CH_G, CH_G))
        dst_cs.append(dst_f[o:o + ec].reshape(NW, ec // NW // CH_S, CH_S))
        ea_c.append(edge_attr[o:o + ec])

    (wn1, bn1), (wn2, bn2) = params["enc_n"]
    (we1, be1), (we2, be2) = params["enc_e"]
    blocks = params["blocks"]
    (wd1, bd1), (wd2, bd2) = params["dec_n"]
    (wg1, bg1), (wg2, bg2), (wg3, bg3) = params["dec_g"]
    t_row = jnp.broadcast_to(params["t"].astype(_f32), (1, H))

    slice_k = [_sc_slice_kernels(ec) for ec in EC_LIST]
    _sc_counts = _sc_counts_kernel()

    ew0 = blocks[0]["edge"][0][0]
    nf, p, q = _node_encode(x, wn1, _row(bn1), wn2, _row(bn2),
                            ew0[0:H], ew0[H:2 * H])

    zeros_acc = jnp.zeros((N, H), _f32)
    cnt = _sc_counts(dst_cnt, jnp.ones((CH_G, H), _f32), zeros_acc)

    (w1, b1), (w2, b2) = blocks[0]["edge"]
    sums0 = []
    eo0_c = []
    for k in range(K):
        g2, sc = slice_k[k]
        pq = g2(dst_c[k], src_c[k], p, q)
        eout = _edge_block0(pq, ea_c[k], we1, _row(be1), we2,
                            _row(be2), w1[2 * H:3 * H], _row(b1),
                            w2, _row(b2))
        eo0_c.append(eout)
        sums0.append(sc(dst_cs[k], eout, zeros_acc))
    (nw1, nb1), (nw2, nb2) = blocks[0]["node"]
    ew1 = blocks[1]["edge"][0][0]
    nf, p, q = _node_block(nf, sums0, cnt, nw1[0:H], nw1[H:2 * H],
                           _row(nb1), nw2, _row(nb2),
                           ew1[0:H], ew1[H:2 * H])

    (w1, b1), (w2, b2) = blocks[1]["edge"]
    sums1 = []
    for k in range(K):
        g2, sc = slice_k[k]
        pq = g2(dst_c[k], src_c[k], p, q)
        eout = _edge_block1(pq, ea_c[k], eo0_c[k], we1, _row(be1), we2,
                            _row(be2), w1[2 * H:3 * H], _row(b1),
                            w2, _row(b2))
        sums1.append(sc(dst_cs[k], eout, zeros_acc))
    (nw1, nb1), (nw2, nb2) = blocks[1]["node"]
    nf = _node_block(nf, sums1, cnt, nw1[0:H], nw1[H:2 * H],
                     _row(nb1), nw2, _row(nb2))

    y, glob = _decode(nf, t_row, wd1, _row(bd1), wd2, _row(bd2),
                      wg1, _row(bg1), wg2, _row(bg2), wg3, _row(bg3))
    return (y, glob)


# speedup vs baseline: 1.0256x; 1.0256x over previous
import functools

import numpy as np

import jax
import jax.numpy as jnp
from jax import lax
from jax.experimental import pallas as pl
from jax.experimental.pallas import tpu as pltpu
from jax.experimental.pallas import tpu_sc as plsc

N = 10000
E = 320000
H = 128
OUT_NODES = 3
OUT_GLOB = 4

NC = 2
NS = 16
NW = NC * NS
EPW = E // NW
CH_G = 80
CH_S = 40
NB = 5
EC_LIST = (192000, 128000)
K = len(EC_LIST)
NCHUNK_CNT = EPW // CH_G

TN = 2000
TE = 2000

_f32 = jnp.float32
_bf16 = jnp.bfloat16
HP = H // 2


def _leaky(v):
    return jnp.where(v >= 0, v, 0.01 * v)


def _dot(a, b):
    return jnp.dot(a, b, preferred_element_type=_f32)


def _node_enc_body(x_ref, w1, b1, w2, b2, wa, wb, nf_ref, p_ref, q_ref):
    h = _leaky(_dot(x_ref[...], w1[...]) + b1[...])
    nf = _dot(h, w2[...]) + b2[...]
    nf_ref[...] = nf
    p_ref[...] = _dot(nf, wa[...])
    q_ref[...] = _dot(nf, wb[...])


def _bdot(a, b):
    return jnp.dot(a.astype(_bf16), b.astype(_bf16),
                   preferred_element_type=_f32)


def _edge_blk0_body(pq_ref, ea_ref, we1, be1, we2, be2,
                    wc, b1, w2, b2, eout_ref, efn_ref):
    he = _leaky(_bdot(ea_ref[...], we1[...]) + be1[...])
    ef = _bdot(he, we2[...]) + be2[...]
    pre = pq_ref[...] + _bdot(ef, wc[...]) + b1[...]
    eo = _bdot(_leaky(pre), w2[...]) + b2[...]
    eout_ref[...] = eo
    efn_ref[...] = eo + ef


def _edge_blk1_body(pq_ref, ef_ref, wc, b1, w2, b2, eout_ref):
    pre = pq_ref[...] + _bdot(ef_ref[...], wc[...]) + b1[...]
    eout_ref[...] = _bdot(_leaky(pre), w2[...]) + b2[...]


def _node_blk_body(nf_ref, *args):
    s_refs = args[:K]
    (c_ref, wa, wb, b1, w2, b2, wpa, wpb,
     nfn_ref, p_ref, q_ref) = args[K:]
    nf = nf_ref[...]
    s = s_refs[0][0] + s_refs[0][1]
    for sr in s_refs[1:]:
        s = s + sr[0] + sr[1]
    cnt = c_ref[0][:, 0:1] + c_ref[1][:, 0:1]
    aggr = s / jnp.maximum(cnt, 1.0)
    h = _leaky(_dot(nf, wa[...]) + _dot(aggr, wb[...]) + b1[...])
    nfn = _dot(h, w2[...]) + b2[...] + nf
    nfn_ref[...] = nfn
    if p_ref is not None:
        p_ref[...] = _dot(nfn, wpa[...])
        q_ref[...] = _dot(nfn, wpb[...])


def _node_blk_last_body(nf_ref, *args):
    s_refs = args[:K]
    c_ref, wa, wb, b1, w2, b2, nfn_ref = args[K:]
    _node_blk_body(nf_ref, *s_refs, c_ref, wa, wb, b1, w2, b2, None, None,
                   nfn_ref, None, None)


def _decode_body(nf_ref, t_ref, wd1, bd1, wd2, bd2,
                 wg1, bg1, wg2, bg2, wg3, bg3,
                 y_ref, g_ref, m_sc, s_sc, v_sc):
    i = pl.program_id(0)
    nf = nf_ref[...]
    h = _leaky(_dot(nf, wd1[...]) + bd1[...])
    y_ref[...] = _dot(h, wd2[...]) + bd2[...]

    alpha = nf * t_ref[...]
    tmax = jnp.max(alpha, axis=0, keepdims=True)

    @pl.when(i == 0)
    def _():
        e = jnp.exp(alpha - tmax)
        m_sc[...] = tmax
        s_sc[...] = jnp.sum(e, axis=0, keepdims=True)
        v_sc[...] = jnp.sum(e * nf, axis=0, keepdims=True)

    @pl.when(i > 0)
    def _():
        m_old = m_sc[...]
        m_new = jnp.maximum(m_old, tmax)
        scale = jnp.exp(m_old - m_new)
        e = jnp.exp(alpha - m_new)
        m_sc[...] = m_new
        s_sc[...] = s_sc[...] * scale + jnp.sum(e, axis=0, keepdims=True)
        v_sc[...] = v_sc[...] * scale + jnp.sum(e * nf, axis=0, keepdims=True)

    @pl.when(i == pl.num_programs(0) - 1)
    def _():
        gin = v_sc[...] / s_sc[...]
        g = _leaky(_dot(gin, wg1[...]) + bg1[...])
        g = _leaky(_dot(g, wg2[...]) + bg2[...])
        g_ref[...] = _dot(g, wg3[...]) + bg3[...]


def _full(shape):
    return pl.BlockSpec(shape, lambda i: (0,) * len(shape))


def _rows(tile, width):
    return pl.BlockSpec((tile, width), lambda i: (i, 0))


def _node_encode(x, w1, b1, w2, b2, wa, wb):
    return pl.pallas_call(
        _node_enc_body,
        grid=(N // TN,),
        in_specs=[_rows(TN, H), _full((H, H)), _full((1, H)), _full((H, H)),
                  _full((1, H)), _full((H, H)), _full((H, H))],
        out_specs=[_rows(TN, H)] * 3,
        out_shape=[jax.ShapeDtypeStruct((N, H), _f32)] * 3,
    )(x, w1, b1, w2, b2, wa, wb)


def _edge_block0(pq, ea, we1, be1, we2, be2, wc, b1, w2, b2):
    f_in = ea.shape[1]
    rows = pq.shape[0]
    return pl.pallas_call(
        _edge_blk0_body,
        grid=(rows // TE,),
        in_specs=[_rows(TE, H), _rows(TE, f_in),
                  _full((f_in, H)), _full((1, H)), _full((H, H)),
                  _full((1, H)), _full((H, H)), _full((1, H)),
                  _full((H, H)), _full((1, H))],
        out_specs=[_rows(TE, H)] * 2,
        out_shape=[jax.ShapeDtypeStruct((rows, H), _f32)] * 2,
    )(pq, ea, we1, be1, we2, be2, wc, b1, w2, b2)


def _edge_block1(pq, ef, wc, b1, w2, b2):
    rows = pq.shape[0]
    return pl.pallas_call(
        _edge_blk1_body,
        grid=(rows // TE,),
        in_specs=[_rows(TE, H), _rows(TE, H), _full((H, H)),
                  _full((1, H)), _full((H, H)), _full((1, H))],
        out_specs=_rows(TE, H),
        out_shape=jax.ShapeDtypeStruct((rows, H), _f32),
    )(pq, ef, wc, b1, w2, b2)


def _node_block(nf, sums_list, cnt, wa, wb, b1, w2, b2, wpa=None, wpb=None):
    sums_spec = pl.BlockSpec((NC, TN, H), lambda i: (0, i, 0))
    common = ([_rows(TN, H)] + [sums_spec] * K + [sums_spec, _full((H, H)),
              _full((H, H)), _full((1, H)), _full((H, H)), _full((1, H))])
    if wpa is None:
        return pl.pallas_call(
            _node_blk_last_body,
            grid=(N // TN,),
            in_specs=common,
            out_specs=_rows(TN, H),
            out_shape=jax.ShapeDtypeStruct((N, H), _f32),
        )(nf, *sums_list, cnt, wa, wb, b1, w2, b2)
    return pl.pallas_call(
        _node_blk_body,
        grid=(N // TN,),
        in_specs=common + [_full((H, H)), _full((H, H))],
        out_specs=[_rows(TN, H)] * 3,
        out_shape=[jax.ShapeDtypeStruct((N, H), _f32)] * 3,
    )(nf, *sums_list, cnt, wa, wb, b1, w2, b2, wpa, wpb)


def _decode(nf, t_row, wd1, bd1, wd2, bd2, wg1, bg1, wg2, bg2, wg3, bg3):
    return pl.pallas_call(
        _decode_body,
        grid=(N // TN,),
        in_specs=[_rows(TN, H), _full((1, H)),
                  _full((H, H)), _full((1, H)), _full((H, OUT_NODES)),
                  _full((1, OUT_NODES)),
                  _full((H, H)), _full((1, H)), _full((H, H)), _full((1, H)),
                  _full((H, OUT_GLOB)), _full((1, OUT_GLOB))],
        out_specs=[_rows(TN, OUT_NODES),
                   pl.BlockSpec((1, OUT_GLOB), lambda i: (0, 0))],
        out_shape=[jax.ShapeDtypeStruct((N, OUT_NODES), _f32),
                   jax.ShapeDtypeStruct((1, OUT_GLOB), _f32)],
        scratch_shapes=[pltpu.VMEM((1, H), _f32)] * 3,
    )(nf, t_row, wd1, bd1, wd2, bd2, wg1, bg1, wg2, bg2, wg3, bg3)


def _ring(nchunk, load_start, load_wait, store_start, store_wait):
    nb = NB
    for k in range(nb - 1):
        load_start(k, k)

    @pl.loop(0, nchunk - nb, step=nb)
    def _(base):
        for b in range(nb):
            i = base + b
            load_wait(i, b)
            store_start(i, b)
            s_n = (b + nb - 1) % nb

            def _drain():
                store_wait(i, s_n)

            if b == 0:
                pl.when(base > 0)(_drain)
            else:
                _drain()
            load_start(i + nb - 1, s_n)

    i0 = nchunk - nb
    load_wait(i0, 0)
    store_start(i0, 0)
    store_wait(i0, nb - 1)
    load_start(nchunk - 1, nb - 1)
    for b in range(1, nb):
        load_wait(i0 + b, b)
        store_start(i0 + b, b)
    for b in range(nb):
        store_wait(i0 + b, b)


@functools.lru_cache(maxsize=None)
def _mesh():
    return plsc.VectorSubcoreMesh(core_axis_name="c", subcore_axis_name="s",
                                  num_cores=NC, num_subcores=NS)


@functools.lru_cache(maxsize=None)
def _sc_slice_kernels(ec):
    mesh = _mesh()
    epw_c = ec // NW
    nchunk_gc = epw_c // CH_G
    nchunk_sc = epw_c // CH_S

    @functools.partial(
        pl.kernel,
        out_type=jax.ShapeDtypeStruct((ec, H), _f32),
        mesh=mesh,
        scratch_types=[pltpu.VMEM((nchunk_gc, CH_G), jnp.int32)] * 2
        + [pltpu.VMEM((CH_G, H), _f32)] * (2 * NB)
        + [pltpu.SemaphoreType.DMA] * (3 * NB),
    )
    def _sc_gather2(dst_hbm, src_hbm, p_hbm, q_hbm, pq_hbm, *rest):
        idxd = rest[0]
        idxs = rest[1]
        bufp = rest[2:2 + NB]
        bufq = rest[2 + NB:2 + 2 * NB]
        sem_p = rest[2 + 2 * NB:2 + 3 * NB]
        sem_q = rest[2 + 3 * NB:2 + 4 * NB]
        sem_w = rest[2 + 4 * NB:2 + 5 * NB]
        wid = lax.axis_index("s") * NC + lax.axis_index("c")
        wbase = wid * epw_c
        pltpu.sync_copy(dst_hbm.at[wid], idxd)
        pltpu.sync_copy(src_hbm.at[wid], idxs)

        def out_sl(i):
            return pq_hbm.at[pl.ds(wbase + i * CH_G, CH_G)]

        def load_start(i, sl):
            pltpu.async_copy(p_hbm.at[idxd.at[i]], bufp[sl], sem_p[sl])
            pltpu.async_copy(q_hbm.at[idxs.at[i]], bufq[sl], sem_q[sl])

        def load_wait(i, sl):
            pltpu.make_async_copy(p_hbm.at[idxd.at[i]], bufp[sl],
                                  sem_p[sl]).wait()
            pltpu.make_async_copy(q_hbm.at[idxs.at[i]], bufq[sl],
                                  sem_q[sl]).wait()

        def w_start(i, sl):
            @pl.loop(0, CH_G)
            def _(r):
                for c in range(H // 16):
                    plsc.addupdate(bufp[sl].at[r, pl.ds(c * 16, 16)],
                                   bufq[sl][r, pl.ds(c * 16, 16)])

            pltpu.async_copy(bufp[sl], out_sl(i), sem_w[sl])

        def w_wait(i, sl):
            pltpu.make_async_copy(bufp[sl], out_sl(i), sem_w[sl]).wait()

        _ring(nchunk_gc, load_start, load_wait, w_start, w_wait)

    @functools.partial(
        pl.kernel,
        out_type=jax.ShapeDtypeStruct((NC, N, H), _f32),
        mesh=mesh,
        scratch_types=[pltpu.VMEM((nchunk_sc, CH_S), jnp.int32)]
        + [pltpu.VMEM((CH_S, H), _f32)] * NB
        + [pltpu.SemaphoreType.DMA] * (2 * NB)
        + [pltpu.VMEM_SHARED((N, H), _f32)],
    )
    def _sc_scatter(dst_hbm, eout_hbm, zeros_hbm, out_hbm, *rest):
        idx_vm = rest[0]
        bufs = rest[1:1 + NB]
        sem_r = rest[1 + NB:1 + 2 * NB]
        sem_a = rest[1 + 2 * NB:1 + 3 * NB]
        acc_sh = rest[1 + 3 * NB]
        cid = lax.axis_index("c")
        sid = lax.axis_index("s")

        @pl.when(sid == 0)
        def _():
            pltpu.sync_copy(zeros_hbm, acc_sh)

        wid = sid * NC + cid
        wbase = wid * epw_c
        pltpu.sync_copy(dst_hbm.at[wid], idx_vm)
        plsc.subcore_barrier()

        def eo_sl(i):
            return eout_hbm.at[pl.ds(wbase + i * CH_S, CH_S)]

        def load_start(i, sl):
            pltpu.async_copy(eo_sl(i), bufs[sl], sem_r[sl])

        def load_wait(i, sl):
            pltpu.make_async_copy(eo_sl(i), bufs[sl], sem_r[sl]).wait()

        def a_start(i, sl):
            pltpu.async_copy(bufs[sl], acc_sh.at[idx_vm.at[i]], sem_a[sl],
                             add=True)

        def a_wait(i, sl):
            pltpu.make_async_copy(bufs[sl], acc_sh.at[idx_vm.at[i]],
                                  sem_a[sl]).wait()

        _ring(nchunk_sc, load_start, load_wait, a_start, a_wait)

        plsc.subcore_barrier()
        rows = (N // NS) // 8 * 8
        rstart = sid * rows
        pltpu.sync_copy(acc_sh.at[pl.ds(rstart, rows)],
                        out_hbm.at[cid].at[pl.ds(rstart, rows)])

        @pl.when(sid == NS - 1)
        def _():
            pltpu.sync_copy(acc_sh.at[pl.ds(NS * rows, N - NS * rows)],
                            out_hbm.at[cid].at[pl.ds(NS * rows, N - NS * rows)])

    return _sc_gather2, _sc_scatter


@functools.lru_cache(maxsize=None)
def _sc_counts_kernel():
    mesh = _mesh()

    @functools.partial(
        pl.kernel,
        out_type=jax.ShapeDtypeStruct((NC, N, H), _f32),
        mesh=mesh,
        scratch_types=[pltpu.VMEM((NCHUNK_CNT, CH_G), jnp.int32),
                       pltpu.VMEM((CH_G, H), _f32),
                       pltpu.SemaphoreType.DMA,
                       pltpu.VMEM_SHARED((N, H), _f32)],
    )
    def _sc_counts(dst_hbm, ones_hbm, zeros_hbm, out_hbm,
                   idx_vm, ones_v, sem_a, acc_sh):
        cid = lax.axis_index("c")
        sid = lax.axis_index("s")

        @pl.when(sid == 0)
        def _():
            pltpu.sync_copy(zeros_hbm, acc_sh)

        wid = sid * NC + cid
        pltpu.sync_copy(dst_hbm.at[wid], idx_vm)
        pltpu.sync_copy(ones_hbm, ones_v)
        plsc.subcore_barrier()
        DEPTH = 8

        @pl.loop(0, NCHUNK_CNT)
        def _(i):
            @pl.when(i >= DEPTH)
            def _():
                pltpu.make_async_copy(ones_v, acc_sh.at[idx_vm.at[i]],
                                      sem_a).wait()

            pltpu.async_copy(ones_v, acc_sh.at[idx_vm.at[i]], sem_a, add=True)

        @pl.loop(0, DEPTH)
        def _(i):
            pltpu.make_async_copy(ones_v, acc_sh.at[idx_vm.at[i]],
                                  sem_a).wait()

        plsc.subcore_barrier()
        rows = (N // NS) // 8 * 8
        rstart = sid * rows
        pltpu.sync_copy(acc_sh.at[pl.ds(rstart, rows)],
                        out_hbm.at[cid].at[pl.ds(rstart, rows)])

        @pl.when(sid == NS - 1)
        def _():
            pltpu.sync_copy(acc_sh.at[pl.ds(NS * rows, N - NS * rows)],
                            out_hbm.at[cid].at[pl.ds(NS * rows, N - NS * rows)])

    return _sc_counts


def _row(b):
    return b.reshape(1, -1)


def kernel(x, edge_index, edge_attr, batch, params):
    del batch
    src_f = edge_index[0]
    dst_f = edge_index[1]
    dst_cnt = dst_f.reshape(NW, NCHUNK_CNT, CH_G)
    offs = [0]
    for ec in EC_LIST:
        offs.append(offs[-1] + ec)
    src_c, dst_c, dst_cs, ea_c = [], [], [], []
    for k, ec in enumerate(EC_LIST):
        o = offs[k]
        src_c.append(src_f[o:o + ec].reshape(NW, ec // NW // CH_G, CH_G))
        dst_c.append(dst_f[o:o + ec].reshape(NW, ec // NW // CH_G, CH_G))
        dst_cs.append(dst_f[o:o + ec].reshape(NW, ec // NW // CH_S, CH_S))
        ea_c.append(edge_attr[o:o + ec])

    (wn1, bn1), (wn2, bn2) = params["enc_n"]
    (we1, be1), (we2, be2) = params["enc_e"]
    blocks = params["blocks"]
    (wd1, bd1), (wd2, bd2) = params["dec_n"]
    (wg1, bg1), (wg2, bg2), (wg3, bg3) = params["dec_g"]
    t_row = jnp.broadcast_to(params["t"].astype(_f32), (1, H))

    slice_k = [_sc_slice_kernels(ec) for ec in EC_LIST]
    _sc_counts = _sc_counts_kernel()

    ew0 = blocks[0]["edge"][0][0]
    nf, p, q = _node_encode(x, wn1, _row(bn1), wn2, _row(bn2),
                            ew0[0:H], ew0[H:2 * H])

    zeros_acc = jnp.zeros((N, H), _f32)
    cnt = _sc_counts(dst_cnt, jnp.ones((CH_G, H), _f32), zeros_acc)

    (w1, b1), (w2, b2) = blocks[0]["edge"]
    sums0 = []
    ef_c = []
    for k in range(K):
        g2, sc = slice_k[k]
        pq = g2(dst_c[k], src_c[k], p, q)
        eout, ef_k = _edge_block0(pq, ea_c[k], we1, _row(be1), we2,
                                  _row(be2), w1[2 * H:3 * H], _row(b1),
                                  w2, _row(b2))
        ef_c.append(ef_k)
        sums0.append(sc(dst_cs[k], eout, zeros_acc))
    (nw1, nb1), (nw2, nb2) = blocks[0]["node"]
    ew1 = blocks[1]["edge"][0][0]
    nf, p, q = _node_block(nf, sums0, cnt, nw1[0:H], nw1[H:2 * H],
                           _row(nb1), nw2, _row(nb2),
                           ew1[0:H], ew1[H:2 * H])

    (w1, b1), (w2, b2) = blocks[1]["edge"]
    sums1 = []
    for k in range(K):
        g2, sc = slice_k[k]
        pq = g2(dst_c[k], src_c[k], p, q)
        eout = _edge_block1(pq, ef_c[k], w1[2 * H:3 * H], _row(b1),
                            w2, _row(b2))
        sums1.append(sc(dst_cs[k], eout, zeros_acc))
    (nw1, nb1), (nw2, nb2) = blocks[1]["node"]
    nf = _node_block(nf, sums1, cnt, nw1[0:H], nw1[H:2 * H],
                     _row(nb1), nw2, _row(nb2))

    y, glob = _decode(nf, t_row, wd1, _row(bd1), wd2, _row(bd2),
                      wg1, _row(bg1), wg2, _row(bg2), wg3, _row(bg3))
    return (y, glob)
